# baseline (speedup 1.0000x reference)
import jax
import jax.numpy as jnp
from jax.experimental import pallas as pl
from jax.experimental.pallas import tpu as pltpu


def _mp_kernel(ni_ref, xt_ref, dkm_ref, recon_ref, fm_ref, dn_ref, tmax_ref,
               t2_ref):
    B, T, C = xt_ref.shape
    A, CK = dkm_ref.shape
    K = CK // C
    PAD = K
    f32 = jnp.float32

    df = dkm_ref[...]
    ss = jnp.sum(df * df, axis=1, keepdims=True)
    dn_ref[...] = df / (jnp.sqrt(ss) + 1e-8)

    NG = T // 8
    patches_all = jnp.concatenate([
        jnp.concatenate(
            [jnp.concatenate([xt_ref[b], jnp.zeros((K - 1, C), f32)],
                             axis=0)[k:k + T, :] for k in range(K)], axis=1)
        for b in range(B)], axis=0)
    res_all = jax.lax.dot_general(
        patches_all, dn_ref[...], (((1,), (1,)), ((), ())),
        preferred_element_type=f32)
    gmax_all = jnp.max(res_all.reshape(B * NG, 8, A), axis=1)
    for b in range(B):
        fm_ref[b, PAD:PAD + T, :] = res_all[b * T:(b + 1) * T, :]
        fm_ref[b, 0:PAD, :] = jnp.zeros((PAD, A), f32)
        fm_ref[b, PAD + T:PAD + T + 2 * K, :] = jnp.zeros((2 * K, A), f32)
        gm = gmax_all[b * NG:(b + 1) * NG, :]
        tmax_ref[b, 0:NG, :] = gm
        tmax_ref[b, NG:NG + 8, :] = jnp.zeros((8, A), f32)
        t2_ref[b, 0:NG // 8, :] = jnp.max(gm.reshape(NG // 8, 8, A), axis=1)
        t2_ref[b, NG // 8:2 * (NG // 8), :] = jnp.zeros((NG // 8, A), f32)

    recon_ref[...] = jnp.zeros_like(recon_ref)

    NG2 = NG // 8
    ir2 = jax.lax.broadcasted_iota(jnp.int32, (NG2, A), 0)
    ga2 = jax.lax.broadcasted_iota(jnp.int32, (NG2, A), 1)
    code2 = ga2 * NG2 + ir2
    ia8 = jax.lax.broadcasted_iota(jnp.int32, (8, A), 1)
    kblk = jax.lax.broadcasted_iota(jnp.int32, (1, CK), 1) // C

    ia_lane = jax.lax.broadcasted_iota(jnp.int32, (1, A), 1)

    def body(i, carry):
        tg_all = t2_ref[:, 0:NG2, :]
        m4 = jnp.max(jnp.max(tg_all, axis=2, keepdims=True), axis=1,
                     keepdims=True)
        sel4 = jnp.min(jnp.min(
            jnp.where(tg_all == m4, code2, jnp.int32(2**31 - 1)),
            axis=2, keepdims=True), axis=1, keepdims=True)
        a_stars, t_stars, vals = [], [], []
        oh_rows = []
        it8c = jax.lax.broadcasted_iota(jnp.int32, (8, 1), 0)
        for b in range(B):
            sel2 = sel4[b, 0, 0]
            a_star = sel2 // NG2
            r2 = sel2 - a_star * NG2
            blk = tmax_ref[b, pl.ds(pl.multiple_of(8 * r2, 8), 8), :]
            colv2 = jnp.max(jnp.where(ia8 == a_star, blk,
                                      jnp.float32(-3e38)),
                            axis=1, keepdims=True)
            m2 = jnp.max(colv2)
            selr = jnp.min(jnp.where(colv2 == m2, it8c, jnp.int32(8)))
            gg = 8 * r2 + jnp.minimum(selr, 7)
            grp = fm_ref[b, pl.ds(pl.multiple_of(8 * gg + PAD, 8), 8), :]
            colv = jnp.max(jnp.where(ia8 == a_star, grp, jnp.float32(-3e38)),
                           axis=1, keepdims=True)
            m8 = jnp.max(colv)
            it8c = jax.lax.broadcasted_iota(jnp.int32, (8, 1), 0)
            sel8 = jnp.min(jnp.where(colv == m8, it8c, jnp.int32(8)))
            a_stars.append(a_star)
            t_stars.append(8 * gg + jnp.minimum(sel8, 7))
            vals.append(m8)
            oh_rows.append((ia_lane == a_star).astype(f32))
        oh4 = jnp.concatenate(oh_rows, axis=0)
        w4 = jax.lax.dot_general(
            oh4, dn_ref[...], (((1,), (0,)), ((), ())),
            preferred_element_type=f32)
        zpad = jnp.zeros((1, (K - 1) * C), f32)
        lagw_rows, wkc_rows = [], []
        for b in range(B):
            wc = (w4[b:b + 1, :] * vals[b]
                  * (kblk < (T - t_stars[b])).astype(f32))
            wkc_rows += [wc[:, k * C:(k + 1) * C] for k in range(K)]
            wpad = jnp.concatenate([zpad, wc, zpad], axis=1)
            lagw_rows += [wpad[:, C * lp:C * lp + CK]
                          for lp in range(2 * K - 1)]
            lagw_rows.append(jnp.zeros((1, CK), f32))
        lagw_all = jnp.concatenate(lagw_rows, axis=0)
        corr_all = jax.lax.dot_general(
            lagw_all, dn_ref[...], (((1,), (1,)), ((), ())),
            preferred_element_type=f32)
        wkc_all = jnp.concatenate(wkc_rows, axis=0)
        r0as, offrs, tas, offts = [], [], [], []
        for b in range(B):
            r0 = t_stars[b] + 1
            r0a = pl.multiple_of((r0 // 8) * 8, 8)
            r0as.append(r0a)
            offrs.append(r0 - r0a)
            ta = pl.multiple_of((t_stars[b] // 8) * 8, 8)
            tas.append(ta)
            offts.append(t_stars[b] - ta)
        i3r = jax.lax.broadcasted_iota(jnp.int32, (B * 3 * K, B * 2 * K), 0)
        i3c = jax.lax.broadcasted_iota(jnp.int32, (B * 3 * K, B * 2 * K), 1)
        rb3 = i3r // (3 * K)
        offv3 = sum(offrs[b] * (rb3 == b).astype(jnp.int32) for b in range(B))
        s3_all = ((i3r % (3 * K) - offv3 == i3c % (2 * K))
                  & (rb3 == i3c // (2 * K))).astype(f32)
        corr24_all = jax.lax.dot_general(
            s3_all, corr_all, (((1,), (0,)), ((), ())),
            preferred_element_type=f32)
        i2r = jax.lax.broadcasted_iota(jnp.int32, (B * 2 * K, B * K), 0)
        i2c = jax.lax.broadcasted_iota(jnp.int32, (B * 2 * K, B * K), 1)
        rb2 = i2r // (2 * K)
        offv2 = sum(offts[b] * (rb2 == b).astype(jnp.int32) for b in range(B))
        s2_all = ((i2r % (2 * K) - offv2 == i2c % K)
                  & (rb2 == i2c // K)).astype(f32)
        add16_all = jax.lax.dot_general(
            s2_all, wkc_all, (((1,), (0,)), ((), ())),
            preferred_element_type=f32)
        it16a = jax.lax.broadcasted_iota(jnp.int32, (2 * K, A), 0)
        for b in range(B):
            r0a = r0as[b]
            fm24 = (fm_ref[b, pl.ds(r0a, 3 * K), :]
                    - corr24_all[b * 3 * K:(b + 1) * 3 * K, :])
            fm_ref[b, pl.ds(r0a, 3 * K), :] = fm24
            jw = r0a // 8 - 1
            ja = pl.multiple_of((jnp.maximum(jw, 0) // 8) * 8, 8)
            off2 = jw - ja
            t16 = tmax_ref[b, pl.ds(ja, 2 * K), :]
            for j in range(3):
                gmj = jnp.max(fm24[8 * j:8 * j + 8, :], axis=0,
                              keepdims=True)
                t16 = jnp.where(it16a == off2 + j, gmj, t16)
            tmax_ref[b, pl.ds(ja, 2 * K), :] = t16
            h0 = jnp.max(t16[0:8, :], axis=0, keepdims=True)
            h1 = jnp.max(t16[8:16, :], axis=0, keepdims=True)
            jb = ja // 8
            itr = jax.lax.broadcasted_iota(jnp.int32, (2 * NG2, A), 0)
            t2blk = t2_ref[b, 0:2 * NG2, :]
            t2_ref[b, 0:2 * NG2, :] = jnp.where(
                itr == jb, h0, jnp.where(itr == jb + 1, h1, t2blk))
            recon_ref[b, pl.ds(tas[b], 2 * K), :] = (
                recon_ref[b, pl.ds(tas[b], 2 * K), :]
                + add16_all[b * 2 * K:(b + 1) * 2 * K, :])
        return carry

    jax.lax.fori_loop(0, ni_ref[0, 0], body, 0)


def kernel(x, d, n_iterations):
    B, C, T = x.shape
    A, _, K = d.shape
    xt = jnp.transpose(x, (0, 2, 1))
    dkm = jnp.transpose(d, (0, 2, 1)).reshape(A, K * C)
    ni = jnp.asarray(n_iterations, jnp.int32).reshape(1, 1)
    recon = pl.pallas_call(
        _mp_kernel,
        out_shape=jax.ShapeDtypeStruct((B, T + K, C), jnp.float32),
        in_specs=[
            pl.BlockSpec(memory_space=pltpu.SMEM),
            pl.BlockSpec(memory_space=pltpu.VMEM),
            pl.BlockSpec(memory_space=pltpu.VMEM),
        ],
        out_specs=pl.BlockSpec(memory_space=pltpu.VMEM),
        scratch_shapes=[
            pltpu.VMEM((B, T + 3 * K, A), jnp.float32),
            pltpu.VMEM((A, K * C), jnp.float32),
            pltpu.VMEM((B, T // 8 + 8, A), jnp.float32),
            pltpu.VMEM((B, 16, A), jnp.float32),
        ],
        compiler_params=pltpu.CompilerParams(
            vmem_limit_bytes=100 * 1024 * 1024),
    )(ni, xt, dkm)
    return jnp.transpose(recon[:, :T, :], (0, 2, 1))


# speedup vs baseline: 1.2077x; 1.2077x over previous
import jax
import jax.numpy as jnp
from jax.experimental import pallas as pl
from jax.experimental.pallas import tpu as pltpu


def _mp_kernel(ni_ref, xt_ref, dkm_ref, recon_ref, fm_ref, dn_ref, tmax_ref):
    B, T, C = xt_ref.shape
    A, CK = dkm_ref.shape
    K = CK // C
    PAD = K
    f32 = jnp.float32

    df = dkm_ref[...]
    ss = jnp.sum(df * df, axis=1, keepdims=True)
    dn_ref[...] = df / (jnp.sqrt(ss) + 1e-8)

    NG = T // 8
    patches_all = jnp.concatenate([
        jnp.concatenate(
            [jnp.concatenate([xt_ref[b], jnp.zeros((K - 1, C), f32)],
                             axis=0)[k:k + T, :] for k in range(K)], axis=1)
        for b in range(B)], axis=0)
    res_all = jax.lax.dot_general(
        patches_all, dn_ref[...], (((1,), (1,)), ((), ())),
        preferred_element_type=f32)
    gmax_all = jnp.max(res_all.reshape(B * NG, 8, A), axis=1)
    for b in range(B):
        fm_ref[b, PAD:PAD + T, :] = res_all[b * T:(b + 1) * T, :]
        fm_ref[b, 0:PAD, :] = jnp.zeros((PAD, A), f32)
        fm_ref[b, PAD + T:PAD + T + 2 * K, :] = jnp.zeros((2 * K, A), f32)
        tmax_ref[b, 0:NG, :] = gmax_all[b * NG:(b + 1) * NG, :]
        tmax_ref[b, NG:NG + 8, :] = jnp.zeros((8, A), f32)

    recon_ref[...] = jnp.zeros_like(recon_ref)

    ig_idx = jax.lax.broadcasted_iota(jnp.int32, (NG, A), 0)
    ga_idx = jax.lax.broadcasted_iota(jnp.int32, (NG, A), 1)
    codeg = ga_idx * NG + ig_idx
    it8 = jax.lax.broadcasted_iota(jnp.int32, (8, A), 0)
    ia8 = jax.lax.broadcasted_iota(jnp.int32, (8, A), 1)
    kblk = jax.lax.broadcasted_iota(jnp.int32, (1, CK), 1) // C

    ia_lane = jax.lax.broadcasted_iota(jnp.int32, (1, A), 1)

    def body(i, carry):
        tg_all = tmax_ref[:, 0:NG, :]
        m4 = jnp.max(jnp.max(tg_all, axis=2, keepdims=True), axis=1,
                     keepdims=True)
        sel4 = jnp.min(jnp.min(
            jnp.where(tg_all == m4, codeg, jnp.int32(2**31 - 1)),
            axis=2, keepdims=True), axis=1, keepdims=True)
        a_stars, t_stars, vals = [], [], []
        oh_rows = []
        for b in range(B):
            selg = sel4[b, 0, 0]
            a_star = selg // NG
            gg = selg - a_star * NG
            grp = fm_ref[b, pl.ds(pl.multiple_of(8 * gg + PAD, 8), 8), :]
            colv = jnp.max(jnp.where(ia8 == a_star, grp, jnp.float32(-3e38)),
                           axis=1, keepdims=True)
            m8 = jnp.max(colv)
            it8c = jax.lax.broadcasted_iota(jnp.int32, (8, 1), 0)
            sel8 = jnp.min(jnp.where(colv == m8, it8c, jnp.int32(8)))
            a_stars.append(a_star)
            t_stars.append(8 * gg + jnp.minimum(sel8, 7))
            vals.append(m8)
            aa = pl.multiple_of((a_star // 8) * 8, 8)
            da8 = dn_ref[pl.ds(aa, 8), :]
            oh_rows.append(pltpu.roll(da8, 8 - (a_star - aa), axis=0)[0:1, :])
        zpad = jnp.zeros((1, (K - 1) * C), f32)
        lagw_rows, wkc_rows = [], []
        for b in range(B):
            wc = (oh_rows[b] * vals[b]
                  * (kblk < (T - t_stars[b])).astype(f32))
            wkc_rows += [wc[:, k * C:(k + 1) * C] for k in range(K)]
            wpad = jnp.concatenate([zpad, wc, zpad], axis=1)
            lagw_rows += [wpad[:, C * lp:C * lp + CK]
                          for lp in range(2 * K - 1)]
            lagw_rows.append(jnp.zeros((1, CK), f32))
        lagw_all = jnp.concatenate(lagw_rows, axis=0)
        corr_all = jax.lax.dot_general(
            lagw_all, dn_ref[...], (((1,), (1,)), ((), ())),
            preferred_element_type=f32)
        wkc_all = jnp.concatenate(wkc_rows, axis=0)
        r0as, offrs, tas, offts = [], [], [], []
        for b in range(B):
            r0 = t_stars[b] + 1
            r0a = pl.multiple_of((r0 // 8) * 8, 8)
            r0as.append(r0a)
            offrs.append(r0 - r0a)
            ta = pl.multiple_of((t_stars[b] // 8) * 8, 8)
            tas.append(ta)
            offts.append(t_stars[b] - ta)
        zrow8 = jnp.zeros((K, A), f32)
        zrow8c = jnp.zeros((K, C), f32)
        it16a = jax.lax.broadcasted_iota(jnp.int32, (2 * K, A), 0)
        for b in range(B):
            r0a = r0as[b]
            corr24 = pltpu.roll(
                jnp.concatenate(
                    [corr_all[b * 2 * K:(b + 1) * 2 * K, :], zrow8], axis=0),
                offrs[b], axis=0)
            add16 = pltpu.roll(
                jnp.concatenate(
                    [wkc_all[b * K:(b + 1) * K, :], zrow8c], axis=0),
                offts[b], axis=0)
            fm24 = fm_ref[b, pl.ds(r0a, 3 * K), :] - corr24
            fm_ref[b, pl.ds(r0a, 3 * K), :] = fm24
            jw = r0a // 8 - 1
            ja = pl.multiple_of((jnp.maximum(jw, 0) // 8) * 8, 8)
            off2 = jw - ja
            t16 = tmax_ref[b, pl.ds(ja, 2 * K), :]
            for j in range(3):
                gmj = jnp.max(fm24[8 * j:8 * j + 8, :], axis=0,
                              keepdims=True)
                t16 = jnp.where(it16a == off2 + j, gmj, t16)
            tmax_ref[b, pl.ds(ja, 2 * K), :] = t16
            recon_ref[b, pl.ds(tas[b], 2 * K), :] = (
                recon_ref[b, pl.ds(tas[b], 2 * K), :]
                + add16)
        return carry

    jax.lax.fori_loop(0, ni_ref[0, 0], body, 0)


def kernel(x, d, n_iterations):
    B, C, T = x.shape
    A, _, K = d.shape
    xt = jnp.transpose(x, (0, 2, 1))
    dkm = jnp.transpose(d, (0, 2, 1)).reshape(A, K * C)
    ni = jnp.asarray(n_iterations, jnp.int32).reshape(1, 1)
    recon = pl.pallas_call(
        _mp_kernel,
        out_shape=jax.ShapeDtypeStruct((B, T + K, C), jnp.float32),
        in_specs=[
            pl.BlockSpec(memory_space=pltpu.SMEM),
            pl.BlockSpec(memory_space=pltpu.VMEM),
            pl.BlockSpec(memory_space=pltpu.VMEM),
        ],
        out_specs=pl.BlockSpec(memory_space=pltpu.VMEM),
        scratch_shapes=[
            pltpu.VMEM((B, T + 3 * K, A), jnp.float32),
            pltpu.VMEM((A, K * C), jnp.float32),
            pltpu.VMEM((B, T // 8 + 8, A), jnp.float32),
        ],
        compiler_params=pltpu.CompilerParams(
            vmem_limit_bytes=100 * 1024 * 1024),
    )(ni, xt, dkm)
    return jnp.transpose(recon[:, :T, :], (0, 2, 1))
